# baseline (speedup 1.0000x reference)
import functools

import jax
import jax.numpy as jnp
from jax import lax
from jax.experimental import pallas as pl
from jax.experimental.pallas import tpu as pltpu
from jax.experimental.pallas import tpu_sc as plsc

_NC = 2
_NW = 32


def _sc_pos_kernel(h, w, nf, row_hbm, col_hbm, out_hbm, buf0, buf1, row_v, sem0, sem1):
    wid = lax.axis_index("s") * _NC + lax.axis_index("c")

    pltpu.sync_copy(col_hbm, buf0.at[:, pl.ds(0, nf)])
    pltpu.sync_copy(col_hbm, buf1.at[:, pl.ds(0, nf)])

    n_iter = (h + _NW - 1) // _NW
    bufs = (buf0, buf1)
    sems = (sem0, sem1)
    nreg = nf // 16

    for t in range(n_iter):
        i = wid + _NW * t
        buf = bufs[t % 2]
        sem = sems[t % 2]

        @pl.when(i < h)
        def _():
            if t >= 2:
                pltpu.make_async_copy(buf, out_hbm.at[i], sem).wait()
            pltpu.sync_copy(row_hbm.at[pl.ds(i, 1), :], row_v)
            regs = [row_v.at[pl.ds(0, 1), pl.ds(16 * c, 16)][...] for c in range(nreg)]

            @pl.loop(0, w)
            def _(r):
                for c in range(nreg):
                    buf.at[pl.ds(r, 1), pl.ds(nf + 16 * c, 16)][...] = regs[c]

            pltpu.async_copy(buf, out_hbm.at[i], sem)

    for t in range(n_iter):
        i = wid + _NW * t

        @pl.when((i < h) & (i + 2 * _NW >= h))
        def _():
            pltpu.make_async_copy(bufs[t % 2], out_hbm.at[i], sems[t % 2]).wait()


def kernel(row_embed, col_embed, bev_h, bev_w):
    h, nf = row_embed.shape
    w, _ = col_embed.shape
    mesh = plsc.VectorSubcoreMesh(core_axis_name="c", subcore_axis_name="s")
    k = pl.kernel(
        functools.partial(_sc_pos_kernel, h, w, nf),
        out_type=jax.ShapeDtypeStruct((h, w, 2 * nf), jnp.float32),
        mesh=mesh,
        scratch_types=[
            pltpu.VMEM((w, 2 * nf), jnp.float32),
            pltpu.VMEM((w, 2 * nf), jnp.float32),
            pltpu.VMEM((1, nf), jnp.float32),
            pltpu.SemaphoreType.DMA,
            pltpu.SemaphoreType.DMA,
        ],
    )
    out = k(row_embed, col_embed)
    return out.reshape(1, h * w, 2 * nf)


# speedup vs baseline: 3.0583x; 3.0583x over previous
import jax
import jax.numpy as jnp
from jax.experimental import pallas as pl


def _pos_body(row_ref, col_ref, out_ref):
    r = row_ref.shape[0]
    nf = row_ref.shape[2]
    w = col_ref.shape[0]
    col = col_ref[...]
    row = row_ref[...]
    out_ref[:, :, 0:nf] = jnp.broadcast_to(col[None, :, :], (r, w, nf))
    out_ref[:, :, nf : 2 * nf] = jnp.broadcast_to(row, (r, w, nf))


def kernel(row_embed, col_embed, bev_h, bev_w):
    h, nf = row_embed.shape
    w, _ = col_embed.shape
    r = 50
    out = pl.pallas_call(
        _pos_body,
        grid=(h // r,),
        in_specs=[
            pl.BlockSpec((r, 1, nf), lambda i: (i, 0, 0)),
            pl.BlockSpec((w, nf), lambda i: (0, 0)),
        ],
        out_specs=pl.BlockSpec((r, w, 2 * nf), lambda i: (i, 0, 0)),
        out_shape=jax.ShapeDtypeStruct((h, w, 2 * nf), jnp.float32),
    )(row_embed.reshape(h, 1, nf), col_embed)
    return out.reshape(1, h * w, 2 * nf)


# speedup vs baseline: 3.2973x; 1.0782x over previous
import jax
import jax.numpy as jnp
from jax.experimental import pallas as pl


def _pos_body(row_ref, col_ref, out_ref):
    r = row_ref.shape[0]
    nf = row_ref.shape[2]
    w = col_ref.shape[0]
    col = col_ref[...]
    row = row_ref[...]
    out_ref[:, :, 0:nf] = jnp.broadcast_to(col[None, :, :], (r, w, nf))
    out_ref[:, :, nf : 2 * nf] = jnp.broadcast_to(row, (r, w, nf))


def kernel(row_embed, col_embed, bev_h, bev_w):
    h, nf = row_embed.shape
    w, _ = col_embed.shape
    r = 20
    out = pl.pallas_call(
        _pos_body,
        grid=(h // r,),
        in_specs=[
            pl.BlockSpec((r, 1, nf), lambda i: (i, 0, 0)),
            pl.BlockSpec((w, nf), lambda i: (0, 0)),
        ],
        out_specs=pl.BlockSpec((r, w, 2 * nf), lambda i: (i, 0, 0)),
        out_shape=jax.ShapeDtypeStruct((h, w, 2 * nf), jnp.float32),
    )(row_embed.reshape(h, 1, nf), col_embed)
    return out.reshape(1, h * w, 2 * nf)
